# initial kernel scaffold (unmeasured)
import jax
import jax.numpy as jnp
from jax import lax
from jax.experimental import pallas as pl
from jax.experimental.pallas import tpu as pltpu

N_DEV = 16
B, SQ, D = 4, 256, 1024
HQ_LOC, DH = 8, 128
KV_COLS = 2 * DH
T = B * SQ
SCALE = 0.08838834764831843
CHUNK = T // N_DEV


def _compute_body(x_ref, wq_ref, wo_ref, wk_ref, wv_ref, out_ref,
                  wk_sl, wv_sl, q_ref, k_ref, v_ref, att_ref, dma_sems):
    i = lax.axis_index("i")
    col0 = i * KV_COLS
    cp_k = pltpu.make_async_copy(
        wk_ref.at[:, pl.ds(col0, KV_COLS)], wk_sl, dma_sems.at[0])
    cp_v = pltpu.make_async_copy(
        wv_ref.at[:, pl.ds(col0, KV_COLS)], wv_sl, dma_sems.at[1])
    cp_k.start()
    cp_v.start()

    x2 = x_ref[...].reshape(T, D)
    q_ref[...] = jnp.dot(x2, wq_ref[...], preferred_element_type=jnp.float32)
    cp_k.wait()
    k_ref[...] = jnp.dot(x2, wk_sl[...], preferred_element_type=jnp.float32)
    cp_v.wait()
    v_ref[...] = jnp.dot(x2, wv_sl[...], preferred_element_type=jnp.float32)

    for b in range(B):
        r0 = b * SQ
        for h in range(HQ_LOC):
            g = h // 4
            qh = q_ref[r0:r0 + SQ, h * DH:(h + 1) * DH]
            kg = k_ref[r0:r0 + SQ, g * DH:(g + 1) * DH]
            vg = v_ref[r0:r0 + SQ, g * DH:(g + 1) * DH]
            s = lax.dot_general(
                qh, kg, (((1,), (1,)), ((), ())),
                preferred_element_type=jnp.float32) * SCALE
            m = jnp.max(s, axis=1, keepdims=True)
            p = jnp.exp(s - m)
            l = jnp.sum(p, axis=1, keepdims=True)
            o = jnp.dot(p, vg, preferred_element_type=jnp.float32) / l
            att_ref[r0:r0 + SQ, h * DH:(h + 1) * DH] = o

    out_ref[...] = jnp.dot(
        att_ref[...], wo_ref[...], preferred_element_type=jnp.float32)


def _allreduce_body(p_ref, out_ref, comm_ref, send_sems, recv_sems):
    i = lax.axis_index("i")
    right = (i + 1) % N_DEV

    out_ref[...] = p_ref[...]

    for s in range(N_DEV - 1):
        c_send = (i - s) % N_DEV
        c_recv = (i - s - 1) % N_DEV
        rdma = pltpu.make_async_remote_copy(
            src_ref=out_ref.at[pl.ds(c_send * CHUNK, CHUNK), :],
            dst_ref=comm_ref.at[s],
            send_sem=send_sems.at[s],
            recv_sem=recv_sems.at[s],
            device_id=(right,),
            device_id_type=pl.DeviceIdType.MESH,
        )
        rdma.start()
        rdma.wait()
        r0 = c_recv * CHUNK
        out_ref[pl.ds(r0, CHUNK), :] = (
            out_ref[pl.ds(r0, CHUNK), :] + comm_ref[s])

    for s in range(N_DEV - 1):
        c = (i + 1 - s) % N_DEV
        rdma = pltpu.make_async_remote_copy(
            src_ref=out_ref.at[pl.ds(c * CHUNK, CHUNK), :],
            dst_ref=out_ref.at[pl.ds(c * CHUNK, CHUNK), :],
            send_sem=send_sems.at[N_DEV - 1 + s],
            recv_sem=recv_sems.at[N_DEV - 1 + s],
            device_id=(right,),
            device_id_type=pl.DeviceIdType.MESH,
        )
        rdma.start()
        rdma.wait()


def kernel(x, Wq, Wo, Wk, Wv):
    partial = pl.pallas_call(
        _compute_body,
        out_shape=jax.ShapeDtypeStruct((T, D), jnp.float32),
        in_specs=[
            pl.BlockSpec(memory_space=pltpu.VMEM),
            pl.BlockSpec(memory_space=pltpu.VMEM),
            pl.BlockSpec(memory_space=pltpu.VMEM),
            pl.BlockSpec(memory_space=pltpu.ANY),
            pl.BlockSpec(memory_space=pltpu.ANY),
        ],
        out_specs=pl.BlockSpec(memory_space=pltpu.VMEM),
        scratch_shapes=[
            pltpu.VMEM((D, KV_COLS), jnp.float32),
            pltpu.VMEM((D, KV_COLS), jnp.float32),
            pltpu.VMEM((T, D), jnp.float32),
            pltpu.VMEM((T, KV_COLS), jnp.float32),
            pltpu.VMEM((T, KV_COLS), jnp.float32),
            pltpu.VMEM((T, D), jnp.float32),
            pltpu.SemaphoreType.DMA((2,)),
        ],
    )(x, Wq, Wo, Wk, Wv)

    reduced = pl.pallas_call(
        _allreduce_body,
        out_shape=jax.ShapeDtypeStruct((T, D), jnp.float32),
        in_specs=[pl.BlockSpec(memory_space=pltpu.VMEM)],
        out_specs=pl.BlockSpec(memory_space=pltpu.VMEM),
        scratch_shapes=[
            pltpu.VMEM((N_DEV - 1, CHUNK, D), jnp.float32),
            pltpu.SemaphoreType.DMA((2 * (N_DEV - 1),)),
            pltpu.SemaphoreType.DMA((2 * (N_DEV - 1),)),
        ],
        compiler_params=pltpu.CompilerParams(collective_id=0),
    )(partial)

    return reduced.reshape(B, SQ, D)


# baseline (device time: 169506 ns/iter reference)
import jax
import jax.numpy as jnp
from jax import lax
from jax.experimental import pallas as pl
from jax.experimental.pallas import tpu as pltpu

N_DEV = 16
B, SQ, D = 4, 256, 1024
HQ_LOC, DH = 8, 128
KV_COLS = 2 * DH
T = B * SQ
SCALE = 0.08838834764831843
CHUNK = T // N_DEV


def _compute_body(x_ref, wq_ref, wo_ref, wk_ref, wv_ref, out_ref,
                  wk_sl, wv_sl, q_ref, k_ref, v_ref, att_ref, dma_sems):
    i = lax.axis_index("i")
    col0 = i * KV_COLS
    cp_k = pltpu.make_async_copy(
        wk_ref.at[:, pl.ds(col0, KV_COLS)], wk_sl, dma_sems.at[0])
    cp_v = pltpu.make_async_copy(
        wv_ref.at[:, pl.ds(col0, KV_COLS)], wv_sl, dma_sems.at[1])
    cp_k.start()
    cp_v.start()

    x2 = x_ref[...].reshape(T, D)
    q_ref[...] = jnp.dot(x2, wq_ref[...], preferred_element_type=jnp.float32)
    cp_k.wait()
    k_ref[...] = jnp.dot(x2, wk_sl[...], preferred_element_type=jnp.float32)
    cp_v.wait()
    v_ref[...] = jnp.dot(x2, wv_sl[...], preferred_element_type=jnp.float32)

    for b in range(B):
        r0 = b * SQ
        for h in range(HQ_LOC):
            g = h // 4
            qh = q_ref[r0:r0 + SQ, h * DH:(h + 1) * DH]
            kg = k_ref[r0:r0 + SQ, g * DH:(g + 1) * DH]
            vg = v_ref[r0:r0 + SQ, g * DH:(g + 1) * DH]
            s = lax.dot_general(
                qh, kg, (((1,), (1,)), ((), ())),
                preferred_element_type=jnp.float32) * SCALE
            m = jnp.max(s, axis=1, keepdims=True)
            p = jnp.exp(s - m)
            l = jnp.sum(p, axis=1, keepdims=True)
            o = jnp.dot(p, vg, preferred_element_type=jnp.float32) / l
            att_ref[r0:r0 + SQ, h * DH:(h + 1) * DH] = o

    out_ref[...] = jnp.dot(
        att_ref[...], wo_ref[...], preferred_element_type=jnp.float32)


def _allreduce_body(p_ref, out_ref, comm_ref, send_sems, recv_sems):
    i = lax.axis_index("i")
    right = (i + 1) % N_DEV

    out_ref[...] = p_ref[...]

    for s in range(N_DEV - 1):
        c_send = (i - s) % N_DEV
        c_recv = (i - s - 1) % N_DEV
        rdma = pltpu.make_async_remote_copy(
            src_ref=out_ref.at[pl.ds(c_send * CHUNK, CHUNK), :],
            dst_ref=comm_ref.at[s],
            send_sem=send_sems.at[s],
            recv_sem=recv_sems.at[s],
            device_id=(right,),
            device_id_type=pl.DeviceIdType.MESH,
        )
        rdma.start()
        rdma.wait()
        r0 = c_recv * CHUNK
        out_ref[pl.ds(r0, CHUNK), :] = (
            out_ref[pl.ds(r0, CHUNK), :] + comm_ref[s])

    for s in range(N_DEV - 1):
        c = (i + 1 - s) % N_DEV
        rdma = pltpu.make_async_remote_copy(
            src_ref=out_ref.at[pl.ds(c * CHUNK, CHUNK), :],
            dst_ref=out_ref.at[pl.ds(c * CHUNK, CHUNK), :],
            send_sem=send_sems.at[N_DEV - 1 + s],
            recv_sem=recv_sems.at[N_DEV - 1 + s],
            device_id=(right,),
            device_id_type=pl.DeviceIdType.MESH,
        )
        rdma.start()
        rdma.wait()


def kernel(x, Wq, Wo, Wk, Wv):
    partial = pl.pallas_call(
        _compute_body,
        out_shape=jax.ShapeDtypeStruct((T, D), jnp.float32),
        in_specs=[
            pl.BlockSpec(memory_space=pltpu.VMEM),
            pl.BlockSpec(memory_space=pltpu.VMEM),
            pl.BlockSpec(memory_space=pltpu.VMEM),
            pl.BlockSpec(memory_space=pltpu.MemorySpace.HBM),
            pl.BlockSpec(memory_space=pltpu.MemorySpace.HBM),
        ],
        out_specs=pl.BlockSpec(memory_space=pltpu.VMEM),
        scratch_shapes=[
            pltpu.VMEM((D, KV_COLS), jnp.float32),
            pltpu.VMEM((D, KV_COLS), jnp.float32),
            pltpu.VMEM((T, D), jnp.float32),
            pltpu.VMEM((T, KV_COLS), jnp.float32),
            pltpu.VMEM((T, KV_COLS), jnp.float32),
            pltpu.VMEM((T, D), jnp.float32),
            pltpu.SemaphoreType.DMA((2,)),
        ],
    )(x, Wq, Wo, Wk, Wv)

    reduced = pl.pallas_call(
        _allreduce_body,
        out_shape=jax.ShapeDtypeStruct((T, D), jnp.float32),
        in_specs=[pl.BlockSpec(memory_space=pltpu.VMEM)],
        out_specs=pl.BlockSpec(memory_space=pltpu.VMEM),
        scratch_shapes=[
            pltpu.VMEM((N_DEV - 1, CHUNK, D), jnp.float32),
            pltpu.SemaphoreType.DMA((2 * (N_DEV - 1),)),
            pltpu.SemaphoreType.DMA((2 * (N_DEV - 1),)),
        ],
    )(partial)

    return reduced.reshape(B, SQ, D)


# device time: 101544 ns/iter; 1.6693x vs baseline; 1.6693x over previous
import jax
import jax.numpy as jnp
from jax import lax
from jax.experimental import pallas as pl
from jax.experimental.pallas import tpu as pltpu

N_DEV = 16
B, SQ, D = 4, 256, 1024
HQ_LOC, DH = 8, 128
KV_COLS = 2 * DH
T = B * SQ
SCALE = 0.08838834764831843


def _compute_body(x_ref, wq_ref, wo_ref, wk_ref, wv_ref, out_ref,
                  wk_sl, wv_sl, q_ref, k_ref, v_ref, att_ref, dma_sems):
    i = lax.axis_index("i")
    col0 = i * KV_COLS
    cp_k = pltpu.make_async_copy(
        wk_ref.at[:, pl.ds(col0, KV_COLS)], wk_sl, dma_sems.at[0])
    cp_v = pltpu.make_async_copy(
        wv_ref.at[:, pl.ds(col0, KV_COLS)], wv_sl, dma_sems.at[1])
    cp_k.start()
    cp_v.start()

    x2 = x_ref[...].reshape(T, D)
    q_ref[...] = jnp.dot(x2, wq_ref[...], preferred_element_type=jnp.float32)
    cp_k.wait()
    k_ref[...] = jnp.dot(x2, wk_sl[...], preferred_element_type=jnp.float32)
    cp_v.wait()
    v_ref[...] = jnp.dot(x2, wv_sl[...], preferred_element_type=jnp.float32)

    for b in range(B):
        r0 = b * SQ
        for h in range(HQ_LOC):
            g = h // 4
            qh = q_ref[r0:r0 + SQ, h * DH:(h + 1) * DH]
            kg = k_ref[r0:r0 + SQ, g * DH:(g + 1) * DH]
            vg = v_ref[r0:r0 + SQ, g * DH:(g + 1) * DH]
            s = lax.dot_general(
                qh, kg, (((1,), (1,)), ((), ())),
                preferred_element_type=jnp.float32) * SCALE
            m = jnp.max(s, axis=1, keepdims=True)
            p = jnp.exp(s - m)
            l = jnp.sum(p, axis=1, keepdims=True)
            o = jnp.dot(p, vg, preferred_element_type=jnp.float32) / l
            att_ref[r0:r0 + SQ, h * DH:(h + 1) * DH] = o

    out_ref[...] = jnp.dot(
        att_ref[...], wo_ref[...], preferred_element_type=jnp.float32)


def _allreduce_body(p_ref, out_ref, c1_ref, c2_ref, c3_ref,
                    send_sems, recv_sems):
    i = lax.axis_index("i")
    Q = i % 4
    Pz = i // 4
    right = Pz * 4 + (Q + 1) % 4
    left = Pz * 4 + (Q - 1) % 4
    pz1 = jnp.bitwise_xor(i, 4)
    pz2 = jnp.bitwise_xor(i, 8)
    t = Pz % 2
    u = Pz // 2

    out_ref[...] = p_ref[...]
    sends = []

    def xfer(src_rows, n_rows, dst, sem_idx, dev):
        rdma = pltpu.make_async_remote_copy(
            src_ref=out_ref.at[pl.ds(src_rows, n_rows), :],
            dst_ref=dst,
            send_sem=send_sems.at[sem_idx],
            recv_sem=recv_sems.at[sem_idx],
            device_id=(dev,),
            device_id_type=pl.DeviceIdType.MESH,
        )
        rdma.start()
        sends.append(rdma)
        return rdma

    def acc(rows, n_rows, buf):
        out_ref[pl.ds(rows, n_rows), :] = (
            out_ref[pl.ds(rows, n_rows), :] + buf)

    for s in range(3):
        rT = xfer(128 * ((Q - s) % 4), 128, c1_ref.at[2 * s],
                  2 * s, right)
        rB = xfer(512 + 128 * ((Q + s) % 4), 128, c1_ref.at[2 * s + 1],
                  2 * s + 1, left)
        rT.wait_recv()
        acc(128 * ((Q - s - 1) % 4), 128, c1_ref[2 * s])
        rB.wait_recv()
        acc(512 + 128 * ((Q + s + 1) % 4), 128, c1_ref[2 * s + 1])

    tT0 = 128 * ((Q + 1) % 4)
    bB0 = 512 + 128 * ((Q - 1) % 4)

    rT = xfer(tT0 + 64 * (1 - t), 64, c2_ref.at[0], 6, pz1)
    rB = xfer(bB0 + 64 * (1 - t), 64, c2_ref.at[1], 7, pz1)
    rT.wait_recv()
    acc(tT0 + 64 * t, 64, c2_ref[0])
    rB.wait_recv()
    acc(bB0 + 64 * t, 64, c2_ref[1])
    rT = xfer(tT0 + 64 * t + 32 * (1 - u), 32, c3_ref.at[0], 8, pz2)
    rB = xfer(bB0 + 64 * t + 32 * (1 - u), 32, c3_ref.at[1], 9, pz2)
    rT.wait_recv()
    acc(tT0 + 64 * t + 32 * u, 32, c3_ref[0])
    rB.wait_recv()
    acc(bB0 + 64 * t + 32 * u, 32, c3_ref[1])

    qT = tT0 + 64 * t + 32 * u
    qB = bB0 + 64 * t + 32 * u
    rT = xfer(qT, 32, out_ref.at[pl.ds(qT, 32), :], 10, pz2)
    rB = xfer(qB, 32, out_ref.at[pl.ds(qB, 32), :], 11, pz2)
    rT.wait_recv()
    rB.wait_recv()
    hT = tT0 + 64 * t
    hB = bB0 + 64 * t
    rT = xfer(hT, 64, out_ref.at[pl.ds(hT, 64), :], 12, pz1)
    rB = xfer(hB, 64, out_ref.at[pl.ds(hB, 64), :], 13, pz1)
    rT.wait_recv()
    rB.wait_recv()

    for s in range(3):
        rowT = 128 * ((Q + 1 - s) % 4)
        rowB = 512 + 128 * ((Q - 1 + s) % 4)
        rT = xfer(rowT, 128, out_ref.at[pl.ds(rowT, 128), :],
                  14 + 2 * s, right)
        rB = xfer(rowB, 128, out_ref.at[pl.ds(rowB, 128), :],
                  15 + 2 * s, left)
        rT.wait_recv()
        rB.wait_recv()

    for rdma in sends:
        rdma.wait_send()


def kernel(x, Wq, Wo, Wk, Wv):
    partial = pl.pallas_call(
        _compute_body,
        out_shape=jax.ShapeDtypeStruct((T, D), jnp.float32),
        in_specs=[
            pl.BlockSpec(memory_space=pltpu.VMEM),
            pl.BlockSpec(memory_space=pltpu.VMEM),
            pl.BlockSpec(memory_space=pltpu.VMEM),
            pl.BlockSpec(memory_space=pltpu.MemorySpace.HBM),
            pl.BlockSpec(memory_space=pltpu.MemorySpace.HBM),
        ],
        out_specs=pl.BlockSpec(memory_space=pltpu.VMEM),
        scratch_shapes=[
            pltpu.VMEM((D, KV_COLS), jnp.float32),
            pltpu.VMEM((D, KV_COLS), jnp.float32),
            pltpu.VMEM((T, D), jnp.float32),
            pltpu.VMEM((T, KV_COLS), jnp.float32),
            pltpu.VMEM((T, KV_COLS), jnp.float32),
            pltpu.VMEM((T, D), jnp.float32),
            pltpu.SemaphoreType.DMA((2,)),
        ],
    )(x, Wq, Wo, Wk, Wv)

    reduced = pl.pallas_call(
        _allreduce_body,
        out_shape=jax.ShapeDtypeStruct((T, D), jnp.float32),
        in_specs=[pl.BlockSpec(memory_space=pltpu.VMEM)],
        out_specs=pl.BlockSpec(memory_space=pltpu.VMEM),
        scratch_shapes=[
            pltpu.VMEM((6, 128, D), jnp.float32),
            pltpu.VMEM((2, 64, D), jnp.float32),
            pltpu.VMEM((2, 32, D), jnp.float32),
            pltpu.SemaphoreType.DMA((20,)),
            pltpu.SemaphoreType.DMA((20,)),
        ],
    )(partial)

    return reduced.reshape(B, SQ, D)


# device time: 89002 ns/iter; 1.9045x vs baseline; 1.1409x over previous
import jax
import jax.numpy as jnp
from jax import lax
from jax.experimental import pallas as pl
from jax.experimental.pallas import tpu as pltpu

N_DEV = 16
B, SQ, D = 4, 256, 1024
HQ_LOC, DH = 8, 128
KV_COLS = 2 * DH
T = B * SQ
SCALE = 0.08838834764831843


def _fused_body(x_ref, wq_ref, wo_ref, wk_ref, wv_ref, out_ref,
                wk_sl, wv_sl, q_ref, k_ref, v_ref, att_ref,
                c1_ref, c2_ref, c3_ref, dma_sems, send_sems, recv_sems):
    i = lax.axis_index("i")
    Q = i % 4
    Pz = i // 4
    right = Pz * 4 + (Q + 1) % 4
    left = Pz * 4 + (Q - 1) % 4
    pz1 = jnp.bitwise_xor(i, 4)
    pz2 = jnp.bitwise_xor(i, 8)
    t = Pz % 2
    u = Pz // 2

    cp_k = pltpu.make_async_copy(
        wk_ref.at[:, pl.ds(i * KV_COLS, KV_COLS)], wk_sl, dma_sems.at[0])
    cp_v = pltpu.make_async_copy(
        wv_ref.at[:, pl.ds(i * KV_COLS, KV_COLS)], wv_sl, dma_sems.at[1])
    cp_k.start()
    cp_v.start()

    x2 = x_ref[...].reshape(T, D)

    def qkv_group(g):
        r0 = 512 * g
        xg = x2[r0:r0 + 512, :]
        q_ref[r0:r0 + 512, :] = jnp.dot(
            xg, wq_ref[...], preferred_element_type=jnp.float32)
        k_ref[r0:r0 + 512, :] = jnp.dot(
            xg, wk_sl[...], preferred_element_type=jnp.float32)
        v_ref[r0:r0 + 512, :] = jnp.dot(
            xg, wv_sl[...], preferred_element_type=jnp.float32)

    def attn_batch(b):
        r0 = b * SQ
        for h in range(HQ_LOC):
            g = h // 4
            qh = q_ref[r0:r0 + SQ, h * DH:(h + 1) * DH]
            kg = k_ref[r0:r0 + SQ, g * DH:(g + 1) * DH]
            vg = v_ref[r0:r0 + SQ, g * DH:(g + 1) * DH]
            s = lax.dot_general(
                qh, kg, (((1,), (1,)), ((), ())),
                preferred_element_type=jnp.float32) * SCALE
            m = jnp.max(s, axis=1, keepdims=True)
            p = jnp.exp(s - m)
            l = jnp.sum(p, axis=1, keepdims=True)
            o = jnp.dot(p, vg, preferred_element_type=jnp.float32) / l
            att_ref[r0:r0 + SQ, h * DH:(h + 1) * DH] = o

    def wo_group(g):
        r0 = 512 * g
        out_ref[r0:r0 + 512, :] = jnp.dot(
            att_ref[r0:r0 + 512, :], wo_ref[...],
            preferred_element_type=jnp.float32)

    sends = []
    pending = []

    def xfer(src_rows, n_rows, dst, sem_idx, dev):
        rdma = pltpu.make_async_remote_copy(
            src_ref=out_ref.at[pl.ds(src_rows, n_rows), :],
            dst_ref=dst,
            send_sem=send_sems.at[sem_idx],
            recv_sem=recv_sems.at[sem_idx],
            device_id=(dev,),
            device_id_type=pl.DeviceIdType.MESH,
        )
        rdma.start()
        sends.append(rdma)
        return rdma

    def step_done(keep=2):
        pending.append(list(sends))
        sends.clear()
        while len(pending) > keep:
            for r in pending.pop(0):
                r.wait_send()

    def acc(rows, n_rows, buf):
        out_ref[pl.ds(rows, n_rows), :] = (
            out_ref[pl.ds(rows, n_rows), :] + buf)

    def rowR(g, c):
        return 512 * g + 64 * (c % 4)

    def rowL(g, c):
        return 512 * g + 256 + 64 * (c % 4)

    def p1_start(g, s):
        off = 20 * g + 2 * s
        rT = xfer(rowR(g, Q - s), 64, c1_ref.at[6 * g + 2 * s], off, right)
        rB = xfer(rowL(g, Q + s), 64, c1_ref.at[6 * g + 2 * s + 1],
                  off + 1, left)
        return rT, rB

    def p1_finish(g, s, rT, rB):
        rT.wait_recv()
        acc(rowR(g, Q - s - 1), 64, c1_ref[6 * g + 2 * s])
        rB.wait_recv()
        acc(rowL(g, Q + s + 1), 64, c1_ref[6 * g + 2 * s + 1])

    def baseR(g):
        return rowR(g, Q + 1)

    def baseL(g):
        return rowL(g, Q - 1)

    def p2_start(g, s):
        off = 20 * g + 6 + 2 * s
        if s == 0:
            rT = xfer(baseR(g) + 32 * (1 - t), 32, c2_ref.at[2 * g],
                      off, pz1)
            rB = xfer(baseL(g) + 32 * (1 - t), 32, c2_ref.at[2 * g + 1],
                      off + 1, pz1)
        else:
            rT = xfer(baseR(g) + 32 * t + 16 * (1 - u), 16,
                      c3_ref.at[2 * g], off, pz2)
            rB = xfer(baseL(g) + 32 * t + 16 * (1 - u), 16,
                      c3_ref.at[2 * g + 1], off + 1, pz2)
        return rT, rB

    def p2_finish(g, s, rT, rB):
        if s == 0:
            rT.wait_recv()
            acc(baseR(g) + 32 * t, 32, c2_ref[2 * g])
            rB.wait_recv()
            acc(baseL(g) + 32 * t, 32, c2_ref[2 * g + 1])
        else:
            rT.wait_recv()
            acc(baseR(g) + 32 * t + 16 * u, 16, c3_ref[2 * g])
            rB.wait_recv()
            acc(baseL(g) + 32 * t + 16 * u, 16, c3_ref[2 * g + 1])

    def p3a_start(g, s):
        off = 20 * g + 10 + 2 * s
        if s == 0:
            qT = baseR(g) + 32 * t + 16 * u
            qB = baseL(g) + 32 * t + 16 * u
            rT = xfer(qT, 16, out_ref.at[pl.ds(qT, 16), :], off, pz2)
            rB = xfer(qB, 16, out_ref.at[pl.ds(qB, 16), :], off + 1, pz2)
        else:
            hT = baseR(g) + 32 * t
            hB = baseL(g) + 32 * t
            rT = xfer(hT, 32, out_ref.at[pl.ds(hT, 32), :], off, pz1)
            rB = xfer(hB, 32, out_ref.at[pl.ds(hB, 32), :], off + 1, pz1)
        return rT, rB

    def p3b_start(g, s):
        off = 20 * g + 14 + 2 * s
        rT_row = rowR(g, Q + 1 - s)
        rB_row = rowL(g, Q - 1 + s)
        rT = xfer(rT_row, 64, out_ref.at[pl.ds(rT_row, 64), :], off, right)
        rB = xfer(rB_row, 64, out_ref.at[pl.ds(rB_row, 64), :],
                  off + 1, left)
        return rT, rB

    def finish_recv(rT, rB):
        rT.wait_recv()
        rB.wait_recv()

    cp_k.wait()
    cp_v.wait()
    qkv_group(0)
    attn_batch(0)
    attn_batch(1)
    wo_group(0)

    r = p1_start(0, 0)
    qkv_group(1)
    p1_finish(0, 0, *r)
    step_done()
    r = p1_start(0, 1)
    attn_batch(2)
    p1_finish(0, 1, *r)
    step_done()
    r = p1_start(0, 2)
    attn_batch(3)
    wo_group(1)
    p1_finish(0, 2, *r)
    step_done()

    a = p2_start(0, 0); b = p1_start(1, 0)
    p2_finish(0, 0, *a); p1_finish(1, 0, *b)
    step_done()

    a = p2_start(0, 1); b = p1_start(1, 1)
    p2_finish(0, 1, *a); p1_finish(1, 1, *b)
    step_done()

    a = p3a_start(0, 0); b = p1_start(1, 2)
    finish_recv(*a); p1_finish(1, 2, *b)
    step_done()

    a = p3a_start(0, 1); b = p2_start(1, 0)
    finish_recv(*a); p2_finish(1, 0, *b)
    step_done()

    a = p3b_start(0, 0); b = p2_start(1, 1)
    finish_recv(*a); p2_finish(1, 1, *b)
    step_done()

    a = p3b_start(0, 1); b = p3a_start(1, 0)
    finish_recv(*a); finish_recv(*b)
    step_done()

    a = p3b_start(0, 2); b = p3a_start(1, 1)
    finish_recv(*a); finish_recv(*b)
    step_done()

    b = p3b_start(1, 0)
    finish_recv(*b)
    step_done()
    b = p3b_start(1, 1)
    finish_recv(*b)
    step_done()
    b = p3b_start(1, 2)
    finish_recv(*b)
    step_done(keep=0)


def kernel(x, Wq, Wo, Wk, Wv):
    reduced = pl.pallas_call(
        _fused_body,
        out_shape=jax.ShapeDtypeStruct((T, D), jnp.float32),
        in_specs=[
            pl.BlockSpec(memory_space=pltpu.VMEM),
            pl.BlockSpec(memory_space=pltpu.VMEM),
            pl.BlockSpec(memory_space=pltpu.VMEM),
            pl.BlockSpec(memory_space=pltpu.MemorySpace.HBM),
            pl.BlockSpec(memory_space=pltpu.MemorySpace.HBM),
        ],
        out_specs=pl.BlockSpec(memory_space=pltpu.VMEM),
        scratch_shapes=[
            pltpu.VMEM((D, KV_COLS), jnp.float32),
            pltpu.VMEM((D, KV_COLS), jnp.float32),
            pltpu.VMEM((T, D), jnp.float32),
            pltpu.VMEM((T, KV_COLS), jnp.float32),
            pltpu.VMEM((T, KV_COLS), jnp.float32),
            pltpu.VMEM((T, D), jnp.float32),
            pltpu.VMEM((12, 64, D), jnp.float32),
            pltpu.VMEM((4, 32, D), jnp.float32),
            pltpu.VMEM((4, 16, D), jnp.float32),
            pltpu.SemaphoreType.DMA((2,)),
            pltpu.SemaphoreType.DMA((40,)),
            pltpu.SemaphoreType.DMA((40,)),
        ],
    )(x, Wq, Wo, Wk, Wv)

    return reduced.reshape(B, SQ, D)


# device time: 69519 ns/iter; 2.4383x vs baseline; 1.2803x over previous
import jax
import jax.numpy as jnp
from jax import lax
from jax.experimental import pallas as pl
from jax.experimental.pallas import tpu as pltpu

N_DEV = 16
B, SQ, D = 4, 256, 1024
HQ_LOC, DH = 8, 128
KV_COLS = 2 * DH
T = B * SQ
SCALE = 0.08838834764831843
WIRE = jnp.float32


def _fused_body(x_ref, wq_ref, wo_ref, wk_ref, wv_ref, out_ref,
                wk_sl, wv_sl, q_ref, k_ref, v_ref, att_ref,
                c1_ref, c2_ref, c3_ref, c4a_ref, c4b_ref, c5_ref, sb_ref,
                dma_sems, send_sems, recv_sems):
    i = lax.axis_index("i")
    Q = i % 4
    Pz = i // 4
    right = Pz * 4 + (Q + 1) % 4
    left = Pz * 4 + (Q - 1) % 4
    pz1 = jnp.bitwise_xor(i, 4)
    pz2 = jnp.bitwise_xor(i, 8)
    t = Pz % 2
    u = Pz // 2

    cp_k = pltpu.make_async_copy(
        wk_ref.at[:, pl.ds(i * KV_COLS, KV_COLS)], wk_sl, dma_sems.at[0])
    cp_v = pltpu.make_async_copy(
        wv_ref.at[:, pl.ds(i * KV_COLS, KV_COLS)], wv_sl, dma_sems.at[1])
    cp_k.start()
    cp_v.start()

    x2 = x_ref[...].reshape(T, D)

    def qkv_group(g):
        r0 = 512 * g
        xg = x2[r0:r0 + 512, :]
        q_ref[r0:r0 + 512, :] = jnp.dot(
            xg, wq_ref[...], preferred_element_type=jnp.float32)
        k_ref[r0:r0 + 512, :] = jnp.dot(
            xg, wk_sl[...], preferred_element_type=jnp.float32)
        v_ref[r0:r0 + 512, :] = jnp.dot(
            xg, wv_sl[...], preferred_element_type=jnp.float32)

    def attn_batch(b):
        r0 = b * SQ
        for h in range(HQ_LOC):
            g = h // 4
            qh = q_ref[r0:r0 + SQ, h * DH:(h + 1) * DH]
            kg = k_ref[r0:r0 + SQ, g * DH:(g + 1) * DH]
            vg = v_ref[r0:r0 + SQ, g * DH:(g + 1) * DH]
            s = lax.dot_general(
                qh, kg, (((1,), (1,)), ((), ())),
                preferred_element_type=jnp.float32) * SCALE
            m = jnp.max(s, axis=1, keepdims=True)
            p = jnp.exp(s - m)
            l = jnp.sum(p, axis=1, keepdims=True)
            o = jnp.dot(p, vg, preferred_element_type=jnp.float32) / l
            att_ref[r0:r0 + SQ, h * DH:(h + 1) * DH] = o

    def wo_group(g):
        r0 = 512 * g
        out_ref[r0:r0 + 512, :] = jnp.dot(
            att_ref[r0:r0 + 512, :], wo_ref[...],
            preferred_element_type=jnp.float32)

    sends = []
    pending = []
    cursor = [0]

    def xfer(src_rows, n_rows, dst, sem_idx, dev):
        s0 = cursor[0]
        h = n_rows // 2
        cursor[0] += h
        sb_ref[s0:s0 + h, :] = pltpu.bitcast(
            out_ref[pl.ds(src_rows, n_rows), :].astype(jnp.bfloat16),
            jnp.float32)
        rdma = pltpu.make_async_remote_copy(
            src_ref=sb_ref.at[pl.ds(s0, h), :],
            dst_ref=dst,
            send_sem=send_sems.at[sem_idx],
            recv_sem=recv_sems.at[sem_idx],
            device_id=(dev,),
            device_id_type=pl.DeviceIdType.MESH,
        )
        rdma.start()
        sends.append(rdma)
        return rdma

    def step_done(keep=2):
        pending.append(list(sends))
        sends.clear()
        while len(pending) > keep:
            for r in pending.pop(0):
                r.wait_send()

    def unpack(buf):
        return pltpu.bitcast(buf, jnp.bfloat16).astype(jnp.float32)

    def acc(rows, n_rows, buf):
        out_ref[pl.ds(rows, n_rows), :] = (
            out_ref[pl.ds(rows, n_rows), :] + unpack(buf))

    def store(rows, n_rows, buf):
        out_ref[pl.ds(rows, n_rows), :] = unpack(buf)

    def rowR(g, c):
        return 512 * g + 64 * (c % 4)

    def rowL(g, c):
        return 512 * g + 256 + 64 * (c % 4)

    def p1_start(g, s):
        off = 20 * g + 2 * s
        rT = xfer(rowR(g, Q - s), 64, c1_ref.at[6 * g + 2 * s], off, right)
        rB = xfer(rowL(g, Q + s), 64, c1_ref.at[6 * g + 2 * s + 1],
                  off + 1, left)
        return rT, rB

    def p1_finish(g, s, rT, rB):
        rT.wait_recv()
        acc(rowR(g, Q - s - 1), 64, c1_ref[6 * g + 2 * s])
        rB.wait_recv()
        acc(rowL(g, Q + s + 1), 64, c1_ref[6 * g + 2 * s + 1])

    def baseR(g):
        return rowR(g, Q + 1)

    def baseL(g):
        return rowL(g, Q - 1)

    def p2_start(g, s):
        off = 20 * g + 6 + 2 * s
        if s == 0:
            rT = xfer(baseR(g) + 32 * (1 - t), 32, c2_ref.at[2 * g],
                      off, pz1)
            rB = xfer(baseL(g) + 32 * (1 - t), 32, c2_ref.at[2 * g + 1],
                      off + 1, pz1)
        else:
            rT = xfer(baseR(g) + 32 * t + 16 * (1 - u), 16,
                      c3_ref.at[2 * g], off, pz2)
            rB = xfer(baseL(g) + 32 * t + 16 * (1 - u), 16,
                      c3_ref.at[2 * g + 1], off + 1, pz2)
        return rT, rB

    def p2_finish(g, s, rT, rB):
        if s == 0:
            rT.wait_recv()
            acc(baseR(g) + 32 * t, 32, c2_ref[2 * g])
            rB.wait_recv()
            acc(baseL(g) + 32 * t, 32, c2_ref[2 * g + 1])
        else:
            rT.wait_recv()
            acc(baseR(g) + 32 * t + 16 * u, 16, c3_ref[2 * g])
            rB.wait_recv()
            acc(baseL(g) + 32 * t + 16 * u, 16, c3_ref[2 * g + 1])

    def p3a_start(g, s):
        off = 20 * g + 10 + 2 * s
        if s == 0:
            qT = baseR(g) + 32 * t + 16 * u
            qB = baseL(g) + 32 * t + 16 * u
            rT = xfer(qT, 16, c4a_ref.at[2 * g], off, pz2)
            rB = xfer(qB, 16, c4a_ref.at[2 * g + 1], off + 1, pz2)
        else:
            hT = baseR(g) + 32 * t
            hB = baseL(g) + 32 * t
            rT = xfer(hT, 32, c4b_ref.at[2 * g], off, pz1)
            rB = xfer(hB, 32, c4b_ref.at[2 * g + 1], off + 1, pz1)
        return rT, rB

    def p3a_finish(g, s, rT, rB):
        if s == 0:
            rT.wait_recv()
            store(baseR(g) + 32 * t + 16 * (1 - u), 16, c4a_ref[2 * g])
            rB.wait_recv()
            store(baseL(g) + 32 * t + 16 * (1 - u), 16, c4a_ref[2 * g + 1])
        else:
            rT.wait_recv()
            store(baseR(g) + 32 * (1 - t), 32, c4b_ref[2 * g])
            rB.wait_recv()
            store(baseL(g) + 32 * (1 - t), 32, c4b_ref[2 * g + 1])

    def p3b_start(g, s):
        off = 20 * g + 14 + 2 * s
        rT = xfer(rowR(g, Q + 1 - s), 64, c5_ref.at[6 * g + 2 * s],
                  off, right)
        rB = xfer(rowL(g, Q - 1 + s), 64, c5_ref.at[6 * g + 2 * s + 1],
                  off + 1, left)
        return rT, rB

    def p3b_finish(g, s, rT, rB):
        rT.wait_recv()
        store(rowR(g, Q - s), 64, c5_ref[6 * g + 2 * s])
        rB.wait_recv()
        store(rowL(g, Q + s), 64, c5_ref[6 * g + 2 * s + 1])

    cp_k.wait()
    cp_v.wait()
    qkv_group(0)
    attn_batch(0)
    attn_batch(1)
    wo_group(0)

    r = p1_start(0, 0)
    qkv_group(1)
    p1_finish(0, 0, *r)
    step_done()
    r = p1_start(0, 1)
    attn_batch(2)
    p1_finish(0, 1, *r)
    step_done()
    r = p1_start(0, 2)
    attn_batch(3)
    wo_group(1)
    p1_finish(0, 2, *r)
    step_done()

    a = p2_start(0, 0); b = p1_start(1, 0)
    p2_finish(0, 0, *a); p1_finish(1, 0, *b)
    step_done()

    a = p2_start(0, 1); b = p1_start(1, 1)
    p2_finish(0, 1, *a); p1_finish(1, 1, *b)
    step_done()

    a = p3a_start(0, 0); b = p1_start(1, 2)
    p3a_finish(0, 0, *a); p1_finish(1, 2, *b)
    step_done()

    a = p3a_start(0, 1); b = p2_start(1, 0)
    p3a_finish(0, 1, *a); p2_finish(1, 0, *b)
    step_done()

    a = p3b_start(0, 0); b = p2_start(1, 1)
    p3b_finish(0, 0, *a); p2_finish(1, 1, *b)
    step_done()

    a = p3b_start(0, 1); b = p3a_start(1, 0)
    p3b_finish(0, 1, *a); p3a_finish(1, 0, *b)
    step_done()

    a = p3b_start(0, 2); b = p3a_start(1, 1)
    p3b_finish(0, 2, *a); p3a_finish(1, 1, *b)
    step_done()

    b = p3b_start(1, 0)
    p3b_finish(1, 0, *b)
    step_done()
    b = p3b_start(1, 1)
    p3b_finish(1, 1, *b)
    step_done()
    b = p3b_start(1, 2)
    p3b_finish(1, 2, *b)
    step_done(keep=0)


def kernel(x, Wq, Wo, Wk, Wv):
    reduced = pl.pallas_call(
        _fused_body,
        out_shape=jax.ShapeDtypeStruct((T, D), jnp.float32),
        in_specs=[
            pl.BlockSpec(memory_space=pltpu.VMEM),
            pl.BlockSpec(memory_space=pltpu.VMEM),
            pl.BlockSpec(memory_space=pltpu.VMEM),
            pl.BlockSpec(memory_space=pltpu.MemorySpace.HBM),
            pl.BlockSpec(memory_space=pltpu.MemorySpace.HBM),
        ],
        out_specs=pl.BlockSpec(memory_space=pltpu.VMEM),
        scratch_shapes=[
            pltpu.VMEM((D, KV_COLS), jnp.float32),
            pltpu.VMEM((D, KV_COLS), jnp.float32),
            pltpu.VMEM((T, D), jnp.float32),
            pltpu.VMEM((T, KV_COLS), jnp.float32),
            pltpu.VMEM((T, KV_COLS), jnp.float32),
            pltpu.VMEM((T, D), jnp.float32),
            pltpu.VMEM((12, 32, D), WIRE),
            pltpu.VMEM((4, 16, D), WIRE),
            pltpu.VMEM((4, 8, D), WIRE),
            pltpu.VMEM((4, 8, D), WIRE),
            pltpu.VMEM((4, 16, D), WIRE),
            pltpu.VMEM((12, 32, D), WIRE),
            pltpu.VMEM((960, D), WIRE),
            pltpu.SemaphoreType.DMA((2,)),
            pltpu.SemaphoreType.DMA((40,)),
            pltpu.SemaphoreType.DMA((40,)),
        ],
    )(x, Wq, Wo, Wk, Wv)

    return reduced.reshape(B, SQ, D)


# device time: 64843 ns/iter; 2.6141x vs baseline; 1.0721x over previous
import jax
import jax.numpy as jnp
from jax import lax
from jax.experimental import pallas as pl
from jax.experimental.pallas import tpu as pltpu

N_DEV = 16
B, SQ, D = 4, 256, 1024
HQ_LOC, DH = 8, 128
KV_COLS = 2 * DH
T = B * SQ
SCALE = 0.08838834764831843
WIRE = jnp.float32


def _fused_body(x_ref, wq_ref, wo_ref, wk_ref, wv_ref, out_ref,
                wk_sl, wv_sl, q_ref, k_ref, v_ref, att_ref,
                c1_ref, c2_ref, c3_ref, c4a_ref, c4b_ref, c5_ref, sb_ref,
                dma_sems, send_sems, recv_sems):
    i = lax.axis_index("i")
    Q = i % 4
    Pz = i // 4
    right = Pz * 4 + (Q + 1) % 4
    left = Pz * 4 + (Q - 1) % 4
    pz1 = jnp.bitwise_xor(i, 4)
    pz2 = jnp.bitwise_xor(i, 8)
    t = Pz % 2
    u = Pz // 2

    cp_k = pltpu.make_async_copy(
        wk_ref.at[:, pl.ds(i * KV_COLS, KV_COLS)], wk_sl, dma_sems.at[0])
    cp_v = pltpu.make_async_copy(
        wv_ref.at[:, pl.ds(i * KV_COLS, KV_COLS)], wv_sl, dma_sems.at[1])
    cp_k.start()
    cp_v.start()

    x2 = x_ref[...].reshape(T, D)

    def qkv_group(g):
        r0 = 512 * g
        xg = x2[r0:r0 + 512, :]
        q_ref[r0:r0 + 512, :] = jnp.dot(
            xg, wq_ref[...], preferred_element_type=jnp.float32)
        k_ref[r0:r0 + 512, :] = jnp.dot(
            xg, wk_sl[...], preferred_element_type=jnp.float32)
        v_ref[r0:r0 + 512, :] = jnp.dot(
            xg, wv_sl[...], preferred_element_type=jnp.float32)

    def attn_batch(b):
        r0 = b * SQ
        for h in range(HQ_LOC):
            g = h // 4
            qh = q_ref[r0:r0 + SQ, h * DH:(h + 1) * DH]
            kg = k_ref[r0:r0 + SQ, g * DH:(g + 1) * DH]
            vg = v_ref[r0:r0 + SQ, g * DH:(g + 1) * DH]
            s = lax.dot_general(
                qh, kg, (((1,), (1,)), ((), ())),
                preferred_element_type=jnp.float32) * SCALE
            m = jnp.max(s, axis=1, keepdims=True)
            p = jnp.exp(s - m)
            l = jnp.sum(p, axis=1, keepdims=True)
            o = jnp.dot(p, vg, preferred_element_type=jnp.float32) / l
            att_ref[r0:r0 + SQ, h * DH:(h + 1) * DH] = o

    def wo_group(g):
        r0 = 512 * g
        out_ref[r0:r0 + 512, :] = jnp.dot(
            att_ref[r0:r0 + 512, :], wo_ref[...],
            preferred_element_type=jnp.float32)

    sends = []
    pending = []
    cursor = [0]

    def xfer(src_rows, n_rows, dst, sem_idx, dev):
        s0 = cursor[0]
        h = n_rows // 2
        cursor[0] += h
        sb_ref[s0:s0 + h, :] = pltpu.bitcast(
            out_ref[pl.ds(src_rows, n_rows), :].astype(jnp.bfloat16),
            jnp.float32)
        rdma = pltpu.make_async_remote_copy(
            src_ref=sb_ref.at[pl.ds(s0, h), :],
            dst_ref=dst,
            send_sem=send_sems.at[sem_idx],
            recv_sem=recv_sems.at[sem_idx],
            device_id=(dev,),
            device_id_type=pl.DeviceIdType.MESH,
        )
        rdma.start()
        sends.append(rdma)
        return rdma

    def step_done(keep=2):
        pending.append(list(sends))
        sends.clear()
        while len(pending) > keep:
            for r in pending.pop(0):
                r.wait_send()

    def unpack(buf):
        return pltpu.bitcast(buf, jnp.bfloat16).astype(jnp.float32)

    def acc(rows, n_rows, buf):
        out_ref[pl.ds(rows, n_rows), :] = (
            out_ref[pl.ds(rows, n_rows), :] + unpack(buf))

    def store(rows, n_rows, buf):
        out_ref[pl.ds(rows, n_rows), :] = unpack(buf)

    def rowR(g, c):
        return 512 * g + 64 * (c % 4)

    def rowL(g, c):
        return 512 * g + 256 + 64 * (c % 4)

    def p1_start(g, s):
        off = 20 * g + 2 * s
        rT = xfer(rowR(g, Q - s), 64, c1_ref.at[6 * g + 2 * s], off, right)
        rB = xfer(rowL(g, Q + s), 64, c1_ref.at[6 * g + 2 * s + 1],
                  off + 1, left)
        return rT, rB

    def p1_finish(g, s, rT, rB):
        rT.wait_recv()
        acc(rowR(g, Q - s - 1), 64, c1_ref[6 * g + 2 * s])
        rB.wait_recv()
        acc(rowL(g, Q + s + 1), 64, c1_ref[6 * g + 2 * s + 1])

    def baseR(g):
        return rowR(g, Q + 1)

    def baseL(g):
        return rowL(g, Q - 1)

    def p2_start(g, s):
        off = 20 * g + 6 + 2 * s
        if s == 0:
            rT = xfer(baseR(g) + 32 * (1 - t), 32, c2_ref.at[2 * g],
                      off, pz1)
            rB = xfer(baseL(g) + 32 * (1 - t), 32, c2_ref.at[2 * g + 1],
                      off + 1, pz1)
        else:
            rT = xfer(baseR(g) + 32 * t + 16 * (1 - u), 16,
                      c3_ref.at[2 * g], off, pz2)
            rB = xfer(baseL(g) + 32 * t + 16 * (1 - u), 16,
                      c3_ref.at[2 * g + 1], off + 1, pz2)
        return rT, rB

    def p2_finish(g, s, rT, rB):
        if s == 0:
            rT.wait_recv()
            acc(baseR(g) + 32 * t, 32, c2_ref[2 * g])
            rB.wait_recv()
            acc(baseL(g) + 32 * t, 32, c2_ref[2 * g + 1])
        else:
            rT.wait_recv()
            acc(baseR(g) + 32 * t + 16 * u, 16, c3_ref[2 * g])
            rB.wait_recv()
            acc(baseL(g) + 32 * t + 16 * u, 16, c3_ref[2 * g + 1])

    def p3a_start(g, s):
        off = 20 * g + 10 + 2 * s
        if s == 0:
            qT = baseR(g) + 32 * t + 16 * u
            qB = baseL(g) + 32 * t + 16 * u
            rT = xfer(qT, 16, c4a_ref.at[2 * g], off, pz2)
            rB = xfer(qB, 16, c4a_ref.at[2 * g + 1], off + 1, pz2)
        else:
            hT = baseR(g) + 32 * t
            hB = baseL(g) + 32 * t
            rT = xfer(hT, 32, c4b_ref.at[2 * g], off, pz1)
            rB = xfer(hB, 32, c4b_ref.at[2 * g + 1], off + 1, pz1)
        return rT, rB

    def p3a_finish(g, s, rT, rB):
        if s == 0:
            rT.wait_recv()
            store(baseR(g) + 32 * t + 16 * (1 - u), 16, c4a_ref[2 * g])
            rB.wait_recv()
            store(baseL(g) + 32 * t + 16 * (1 - u), 16, c4a_ref[2 * g + 1])
        else:
            rT.wait_recv()
            store(baseR(g) + 32 * (1 - t), 32, c4b_ref[2 * g])
            rB.wait_recv()
            store(baseL(g) + 32 * (1 - t), 32, c4b_ref[2 * g + 1])

    def p3b_start(g, s):
        off = 20 * g + 14 + 2 * s
        rT = xfer(rowR(g, Q + 1 - s), 64, c5_ref.at[6 * g + 2 * s],
                  off, right)
        rB = xfer(rowL(g, Q - 1 + s), 64, c5_ref.at[6 * g + 2 * s + 1],
                  off + 1, left)
        return rT, rB

    def p3b_finish(g, s, rT, rB):
        rT.wait_recv()
        store(rowR(g, Q - s), 64, c5_ref[6 * g + 2 * s])
        rB.wait_recv()
        store(rowL(g, Q + s), 64, c5_ref[6 * g + 2 * s + 1])

    cp_k.wait()
    cp_v.wait()
    qkv_group(0)
    attn_batch(0)
    attn_batch(1)
    wo_group(0)

    barrier_sem = pltpu.get_barrier_semaphore()
    for nbr in (right, left, pz1, pz2):
        pl.semaphore_signal(
            barrier_sem, inc=1,
            device_id=(nbr,), device_id_type=pl.DeviceIdType.MESH)
    pl.semaphore_wait(barrier_sem, 4)

    r = p1_start(0, 0)
    qkv_group(1)
    p1_finish(0, 0, *r)
    step_done()
    r = p1_start(0, 1)
    attn_batch(2)
    p1_finish(0, 1, *r)
    step_done()
    r = p1_start(0, 2)
    attn_batch(3)
    wo_group(1)
    p1_finish(0, 2, *r)
    step_done()

    a = p2_start(0, 0); b = p1_start(1, 0)
    p2_finish(0, 0, *a); p1_finish(1, 0, *b)
    step_done()

    a = p2_start(0, 1); b = p1_start(1, 1)
    p2_finish(0, 1, *a); p1_finish(1, 1, *b)
    step_done()

    a = p3a_start(0, 0); b = p1_start(1, 2)
    p3a_finish(0, 0, *a); p1_finish(1, 2, *b)
    step_done()

    a = p3a_start(0, 1); b = p2_start(1, 0)
    p3a_finish(0, 1, *a); p2_finish(1, 0, *b)
    step_done()

    a = p3b_start(0, 0); b = p2_start(1, 1)
    p3b_finish(0, 0, *a); p2_finish(1, 1, *b)
    step_done()

    a = p3b_start(0, 1); b = p3a_start(1, 0)
    p3b_finish(0, 1, *a); p3a_finish(1, 0, *b)
    step_done()

    a = p3b_start(0, 2); b = p3a_start(1, 1)
    p3b_finish(0, 2, *a); p3a_finish(1, 1, *b)
    step_done()

    b = p3b_start(1, 0)
    p3b_finish(1, 0, *b)
    step_done()
    b = p3b_start(1, 1)
    p3b_finish(1, 1, *b)
    step_done()
    b = p3b_start(1, 2)
    p3b_finish(1, 2, *b)
    step_done(keep=0)


def kernel(x, Wq, Wo, Wk, Wv):
    reduced = pl.pallas_call(
        _fused_body,
        out_shape=jax.ShapeDtypeStruct((T, D), jnp.float32),
        in_specs=[
            pl.BlockSpec(memory_space=pltpu.VMEM),
            pl.BlockSpec(memory_space=pltpu.VMEM),
            pl.BlockSpec(memory_space=pltpu.VMEM),
            pl.BlockSpec(memory_space=pltpu.MemorySpace.HBM),
            pl.BlockSpec(memory_space=pltpu.MemorySpace.HBM),
        ],
        out_specs=pl.BlockSpec(memory_space=pltpu.VMEM),
        scratch_shapes=[
            pltpu.VMEM((D, KV_COLS), jnp.float32),
            pltpu.VMEM((D, KV_COLS), jnp.float32),
            pltpu.VMEM((T, D), jnp.float32),
            pltpu.VMEM((T, KV_COLS), jnp.float32),
            pltpu.VMEM((T, KV_COLS), jnp.float32),
            pltpu.VMEM((T, D), jnp.float32),
            pltpu.VMEM((12, 32, D), WIRE),
            pltpu.VMEM((4, 16, D), WIRE),
            pltpu.VMEM((4, 8, D), WIRE),
            pltpu.VMEM((4, 8, D), WIRE),
            pltpu.VMEM((4, 16, D), WIRE),
            pltpu.VMEM((12, 32, D), WIRE),
            pltpu.VMEM((960, D), WIRE),
            pltpu.SemaphoreType.DMA((2,)),
            pltpu.SemaphoreType.DMA((40,)),
            pltpu.SemaphoreType.DMA((40,)),
        ],
        compiler_params=pltpu.CompilerParams(collective_id=0),
    )(x, Wq, Wo, Wk, Wv)

    return reduced.reshape(B, SQ, D)


# device time: 64769 ns/iter; 2.6171x vs baseline; 1.0011x over previous
import jax
import jax.numpy as jnp
from jax import lax
from jax.experimental import pallas as pl
from jax.experimental.pallas import tpu as pltpu

N_DEV = 16
B, SQ, D = 4, 256, 1024
HQ_LOC, DH = 8, 128
KV_COLS = 2 * DH
T = B * SQ
SCALE = 0.08838834764831843
WIRE = jnp.float32


def _fused_body(x_ref, wq_ref, wo_ref, wk_ref, wv_ref, out_ref,
                wk_sl, wv_sl, q_ref, k_ref, v_ref, att_ref,
                c1_ref, c2_ref, c3_ref, c4a_ref, c4b_ref, c5_ref, sb_ref,
                dma_sems, send_sems, recv_sems):
    i = lax.axis_index("i")
    Q = i % 4
    Pz = i // 4
    right = Pz * 4 + (Q + 1) % 4
    left = Pz * 4 + (Q - 1) % 4
    pz1 = jnp.bitwise_xor(i, 4)
    pz2 = jnp.bitwise_xor(i, 8)
    t = Pz % 2
    u = Pz // 2

    cp_k = pltpu.make_async_copy(
        wk_ref.at[:, pl.ds(i * KV_COLS, KV_COLS)], wk_sl, dma_sems.at[0])
    cp_v = pltpu.make_async_copy(
        wv_ref.at[:, pl.ds(i * KV_COLS, KV_COLS)], wv_sl, dma_sems.at[1])
    cp_k.start()
    cp_v.start()

    x2 = x_ref[...].reshape(T, D)

    def qkv_group(g):
        r0 = 512 * g
        xg = x2[r0:r0 + 512, :]
        q_ref[r0:r0 + 512, :] = jnp.dot(
            xg, wq_ref[...], preferred_element_type=jnp.float32)
        k_ref[r0:r0 + 512, :] = jnp.dot(
            xg, wk_sl[...], preferred_element_type=jnp.float32)
        v_ref[r0:r0 + 512, :] = jnp.dot(
            xg, wv_sl[...], preferred_element_type=jnp.float32)

    def attn_batch(b):
        r0 = b * SQ
        for h in range(HQ_LOC):
            g = h // 4
            qh = q_ref[r0:r0 + SQ, h * DH:(h + 1) * DH]
            kg = k_ref[r0:r0 + SQ, g * DH:(g + 1) * DH]
            vg = v_ref[r0:r0 + SQ, g * DH:(g + 1) * DH]
            s = lax.dot_general(
                qh, kg, (((1,), (1,)), ((), ())),
                preferred_element_type=jnp.float32) * SCALE
            m = jnp.max(s, axis=1, keepdims=True)
            p = jnp.exp(s - m)
            l = jnp.sum(p, axis=1, keepdims=True)
            o = jnp.dot(p, vg, preferred_element_type=jnp.float32) / l
            att_ref[r0:r0 + SQ, h * DH:(h + 1) * DH] = o

    def wo_group(g):
        r0 = 512 * g
        out_ref[r0:r0 + 512, :] = jnp.dot(
            att_ref[r0:r0 + 512, :], wo_ref[...],
            preferred_element_type=jnp.float32)

    sends = []
    pending = []
    cursor = [0]

    def xfer(src_rows, n_rows, dst, sem_idx, dev, packed_src=None):
        if packed_src is None:
            s0 = cursor[0]
            h = n_rows // 2
            cursor[0] += h
            sb_ref[s0:s0 + h, :] = pltpu.bitcast(
                out_ref[pl.ds(src_rows, n_rows), :].astype(jnp.bfloat16),
                jnp.float32)
            packed_src = sb_ref.at[pl.ds(s0, h), :]
        rdma = pltpu.make_async_remote_copy(
            src_ref=packed_src,
            dst_ref=dst,
            send_sem=send_sems.at[sem_idx],
            recv_sem=recv_sems.at[sem_idx],
            device_id=(dev,),
            device_id_type=pl.DeviceIdType.MESH,
        )
        rdma.start()
        sends.append(rdma)
        return rdma

    def step_done(keep=2):
        pending.append(list(sends))
        sends.clear()
        while len(pending) > keep:
            for r in pending.pop(0):
                r.wait_send()

    def unpack(buf):
        return pltpu.bitcast(buf, jnp.bfloat16).astype(jnp.float32)

    def acc(rows, n_rows, buf):
        out_ref[pl.ds(rows, n_rows), :] = (
            out_ref[pl.ds(rows, n_rows), :] + unpack(buf))

    def store(rows, n_rows, buf):
        out_ref[pl.ds(rows, n_rows), :] = unpack(buf)

    def rowR(g, c):
        return 512 * g + 64 * (c % 4)

    def rowL(g, c):
        return 512 * g + 256 + 64 * (c % 4)

    def p1_start(g, s):
        off = 20 * g + 2 * s
        rT = xfer(rowR(g, Q - s), 64, c1_ref.at[6 * g + 2 * s], off, right)
        rB = xfer(rowL(g, Q + s), 64, c1_ref.at[6 * g + 2 * s + 1],
                  off + 1, left)
        return rT, rB

    def p1_finish(g, s, rT, rB):
        rT.wait_recv()
        acc(rowR(g, Q - s - 1), 64, c1_ref[6 * g + 2 * s])
        rB.wait_recv()
        acc(rowL(g, Q + s + 1), 64, c1_ref[6 * g + 2 * s + 1])

    def baseR(g):
        return rowR(g, Q + 1)

    def baseL(g):
        return rowL(g, Q - 1)

    def p2_start(g, s):
        off = 20 * g + 6 + 2 * s
        if s == 0:
            rT = xfer(baseR(g) + 32 * (1 - t), 32, c2_ref.at[2 * g],
                      off, pz1)
            rB = xfer(baseL(g) + 32 * (1 - t), 32, c2_ref.at[2 * g + 1],
                      off + 1, pz1)
        else:
            rT = xfer(baseR(g) + 32 * t + 16 * (1 - u), 16,
                      c3_ref.at[2 * g], off, pz2)
            rB = xfer(baseL(g) + 32 * t + 16 * (1 - u), 16,
                      c3_ref.at[2 * g + 1], off + 1, pz2)
        return rT, rB

    def p2_finish(g, s, rT, rB):
        if s == 0:
            rT.wait_recv()
            acc(baseR(g) + 32 * t, 32, c2_ref[2 * g])
            rB.wait_recv()
            acc(baseL(g) + 32 * t, 32, c2_ref[2 * g + 1])
        else:
            rT.wait_recv()
            acc(baseR(g) + 32 * t + 16 * u, 16, c3_ref[2 * g])
            rB.wait_recv()
            acc(baseL(g) + 32 * t + 16 * u, 16, c3_ref[2 * g + 1])

    def p3a_start(g, s):
        off = 20 * g + 10 + 2 * s
        if s == 0:
            qT = baseR(g) + 32 * t + 16 * u
            qB = baseL(g) + 32 * t + 16 * u
            rT = xfer(qT, 16, c4a_ref.at[2 * g], off, pz2)
            rB = xfer(qB, 16, c4a_ref.at[2 * g + 1], off + 1, pz2)
        else:
            hT = baseR(g) + 32 * t
            hB = baseL(g) + 32 * t
            rT = xfer(hT, 32, c4b_ref.at[2 * g], off, pz1)
            rB = xfer(hB, 32, c4b_ref.at[2 * g + 1], off + 1, pz1)
        return rT, rB

    def p3a_finish(g, s, rT, rB):
        if s == 0:
            rT.wait_recv()
            store(baseR(g) + 32 * t + 16 * (1 - u), 16, c4a_ref[2 * g])
            rB.wait_recv()
            store(baseL(g) + 32 * t + 16 * (1 - u), 16, c4a_ref[2 * g + 1])
        else:
            rT.wait_recv()
            store(baseR(g) + 32 * (1 - t), 32, c4b_ref[2 * g])
            rB.wait_recv()
            store(baseL(g) + 32 * (1 - t), 32, c4b_ref[2 * g + 1])

    def p3b_start(g, s):
        off = 20 * g + 14 + 2 * s
        if s == 0:
            rT = xfer(rowR(g, Q + 1), 64, c5_ref.at[6 * g], off, right)
            rB = xfer(rowL(g, Q - 1), 64, c5_ref.at[6 * g + 1],
                      off + 1, left)
        else:
            rT = xfer(0, 64, c5_ref.at[6 * g + 2 * s], off, right,
                      packed_src=c5_ref.at[6 * g + 2 * (s - 1)])
            rB = xfer(0, 64, c5_ref.at[6 * g + 2 * s + 1], off + 1, left,
                      packed_src=c5_ref.at[6 * g + 2 * (s - 1) + 1])
        return rT, rB

    def p3b_finish(g, s, rT, rB):
        rT.wait_recv()
        store(rowR(g, Q - s), 64, c5_ref[6 * g + 2 * s])
        rB.wait_recv()
        store(rowL(g, Q + s), 64, c5_ref[6 * g + 2 * s + 1])

    cp_k.wait()
    cp_v.wait()
    qkv_group(0)
    attn_batch(0)
    attn_batch(1)
    wo_group(0)

    barrier_sem = pltpu.get_barrier_semaphore()
    for nbr in (right, left, pz1, pz2):
        pl.semaphore_signal(
            barrier_sem, inc=1,
            device_id=(nbr,), device_id_type=pl.DeviceIdType.MESH)
    pl.semaphore_wait(barrier_sem, 4)

    r = p1_start(0, 0)
    qkv_group(1)
    p1_finish(0, 0, *r)
    step_done()
    r = p1_start(0, 1)
    attn_batch(2)
    p1_finish(0, 1, *r)
    step_done()
    r = p1_start(0, 2)
    attn_batch(3)
    wo_group(1)
    p1_finish(0, 2, *r)
    step_done()

    a = p2_start(0, 0); b = p1_start(1, 0)
    p2_finish(0, 0, *a); p1_finish(1, 0, *b)
    step_done()

    a = p2_start(0, 1); b = p1_start(1, 1)
    p2_finish(0, 1, *a); p1_finish(1, 1, *b)
    step_done()

    a = p3a_start(0, 0); b = p1_start(1, 2)
    p3a_finish(0, 0, *a); p1_finish(1, 2, *b)
    step_done()

    a = p3a_start(0, 1); b = p2_start(1, 0)
    p3a_finish(0, 1, *a); p2_finish(1, 0, *b)
    step_done()

    a = p3b_start(0, 0); b = p2_start(1, 1)
    p3b_finish(0, 0, *a); p2_finish(1, 1, *b)
    step_done()

    a = p3b_start(0, 1); b = p3a_start(1, 0)
    p3b_finish(0, 1, *a); p3a_finish(1, 0, *b)
    step_done()

    a = p3b_start(0, 2); b = p3a_start(1, 1)
    p3b_finish(0, 2, *a); p3a_finish(1, 1, *b)
    step_done()

    b = p3b_start(1, 0)
    p3b_finish(1, 0, *b)
    step_done()
    b = p3b_start(1, 1)
    p3b_finish(1, 1, *b)
    step_done()
    b = p3b_start(1, 2)
    p3b_finish(1, 2, *b)
    step_done(keep=0)


def kernel(x, Wq, Wo, Wk, Wv):
    reduced = pl.pallas_call(
        _fused_body,
        out_shape=jax.ShapeDtypeStruct((T, D), jnp.float32),
        in_specs=[
            pl.BlockSpec(memory_space=pltpu.VMEM),
            pl.BlockSpec(memory_space=pltpu.VMEM),
            pl.BlockSpec(memory_space=pltpu.VMEM),
            pl.BlockSpec(memory_space=pltpu.MemorySpace.HBM),
            pl.BlockSpec(memory_space=pltpu.MemorySpace.HBM),
        ],
        out_specs=pl.BlockSpec(memory_space=pltpu.VMEM),
        scratch_shapes=[
            pltpu.VMEM((D, KV_COLS), jnp.float32),
            pltpu.VMEM((D, KV_COLS), jnp.float32),
            pltpu.VMEM((T, D), jnp.float32),
            pltpu.VMEM((T, KV_COLS), jnp.float32),
            pltpu.VMEM((T, KV_COLS), jnp.float32),
            pltpu.VMEM((T, D), jnp.float32),
            pltpu.VMEM((12, 32, D), WIRE),
            pltpu.VMEM((4, 16, D), WIRE),
            pltpu.VMEM((4, 8, D), WIRE),
            pltpu.VMEM((4, 8, D), WIRE),
            pltpu.VMEM((4, 16, D), WIRE),
            pltpu.VMEM((12, 32, D), WIRE),
            pltpu.VMEM((960, D), WIRE),
            pltpu.SemaphoreType.DMA((2,)),
            pltpu.SemaphoreType.DMA((40,)),
            pltpu.SemaphoreType.DMA((40,)),
        ],
        compiler_params=pltpu.CompilerParams(collective_id=0),
    )(x, Wq, Wo, Wk, Wv)

    return reduced.reshape(B, SQ, D)


# device time: 59948 ns/iter; 2.8276x vs baseline; 1.0804x over previous
import jax
import jax.numpy as jnp
from jax import lax
from jax.experimental import pallas as pl
from jax.experimental.pallas import tpu as pltpu

N_DEV = 16
B, SQ, D = 4, 256, 1024
HQ_LOC, DH = 8, 128
KV_COLS = 2 * DH
T = B * SQ
SCALE = 0.08838834764831843
WIRE = jnp.float32


def _fused_body(x_ref, wq_ref, wo_ref, wk_ref, wv_ref, out_ref,
                wk_sl, wv_sl, q_ref, k_ref, v_ref, att_ref,
                c1_ref, c2_ref, c3_ref, c4a_ref, c4b_ref, c5_ref, sb_ref,
                dma_sems, send_sems, recv_sems):
    i = lax.axis_index("i")
    Q = i % 4
    Pz = i // 4
    px = jnp.bitwise_and(jnp.bitwise_xor(Q, Q // 2), 1)
    py = Q // 2
    xp = Pz * 4 + jnp.bitwise_xor(Q, 1)
    yp = Pz * 4 + jnp.bitwise_xor(Q, 3)
    pz1 = jnp.bitwise_xor(i, 4)
    pz2 = jnp.bitwise_xor(i, 8)
    t = Pz % 2
    u = Pz // 2

    cp_k = pltpu.make_async_copy(
        wk_ref.at[:, pl.ds(i * KV_COLS, KV_COLS)], wk_sl, dma_sems.at[0])
    cp_v = pltpu.make_async_copy(
        wv_ref.at[:, pl.ds(i * KV_COLS, KV_COLS)], wv_sl, dma_sems.at[1])
    cp_k.start()
    cp_v.start()

    x2 = x_ref[...].reshape(T, D)

    def qkv_group(g):
        r0 = 512 * g
        xg = x2[r0:r0 + 512, :]
        q_ref[r0:r0 + 512, :] = jnp.dot(
            xg, wq_ref[...], preferred_element_type=jnp.float32)
        k_ref[r0:r0 + 512, :] = jnp.dot(
            xg, wk_sl[...], preferred_element_type=jnp.float32)
        v_ref[r0:r0 + 512, :] = jnp.dot(
            xg, wv_sl[...], preferred_element_type=jnp.float32)

    def attn_batch(b):
        r0 = b * SQ
        for h in range(HQ_LOC):
            g = h // 4
            qh = q_ref[r0:r0 + SQ, h * DH:(h + 1) * DH]
            kg = k_ref[r0:r0 + SQ, g * DH:(g + 1) * DH]
            vg = v_ref[r0:r0 + SQ, g * DH:(g + 1) * DH]
            s = lax.dot_general(
                qh, kg, (((1,), (1,)), ((), ())),
                preferred_element_type=jnp.float32) * SCALE
            m = jnp.max(s, axis=1, keepdims=True)
            p = jnp.exp(s - m)
            l = jnp.sum(p, axis=1, keepdims=True)
            o = jnp.dot(p, vg, preferred_element_type=jnp.float32) / l
            att_ref[r0:r0 + SQ, h * DH:(h + 1) * DH] = o

    def wo_group(g):
        r0 = 512 * g
        out_ref[r0:r0 + 512, :] = jnp.dot(
            att_ref[r0:r0 + 512, :], wo_ref[...],
            preferred_element_type=jnp.float32)

    sends = []
    pending = []
    cursor = [0]

    def xfer(src_rows, n_rows, dst, sem_idx, dev, packed_src=None):
        if packed_src is None:
            s0 = cursor[0]
            h = n_rows // 2
            cursor[0] += h
            sb_ref[s0:s0 + h, :] = pltpu.bitcast(
                out_ref[pl.ds(src_rows, n_rows), :].astype(jnp.bfloat16),
                jnp.float32)
            packed_src = sb_ref.at[pl.ds(s0, h), :]
        rdma = pltpu.make_async_remote_copy(
            src_ref=packed_src,
            dst_ref=dst,
            send_sem=send_sems.at[sem_idx],
            recv_sem=recv_sems.at[sem_idx],
            device_id=(dev,),
            device_id_type=pl.DeviceIdType.MESH,
        )
        rdma.start()
        sends.append(rdma)
        return rdma

    def step_done(keep=2):
        pending.append(list(sends))
        sends.clear()
        while len(pending) > keep:
            for r in pending.pop(0):
                r.wait_send()

    def unpack(buf):
        return pltpu.bitcast(buf, jnp.bfloat16).astype(jnp.float32)

    def acc(rows, n_rows, buf):
        out_ref[pl.ds(rows, n_rows), :] = (
            out_ref[pl.ds(rows, n_rows), :] + unpack(buf))

    def store(rows, n_rows, buf):
        out_ref[pl.ds(rows, n_rows), :] = unpack(buf)

    def p1_start(g, s):
        base = 512 * g
        off = 16 * g + 2 * s
        if s == 0:
            rT = xfer(base + 128 * (1 - px), 128,
                      c1_ref.at[4 * g, pl.ds(0, 64)], off, xp)
            rB = xfer(base + 256 + 128 * (1 - py), 128,
                      c1_ref.at[4 * g + 1, pl.ds(0, 64)], off + 1, yp)
        else:
            rT = xfer(base + 128 * px + 64 * (1 - py), 64,
                      c1_ref.at[4 * g + 2, pl.ds(0, 32)], off, yp)
            rB = xfer(base + 256 + 128 * py + 64 * (1 - px), 64,
                      c1_ref.at[4 * g + 3, pl.ds(0, 32)], off + 1, xp)
        return rT, rB

    def p1_finish(g, s, rT, rB):
        base = 512 * g
        if s == 0:
            rT.wait_recv()
            acc(base + 128 * px, 128, c1_ref[4 * g, :64])
            rB.wait_recv()
            acc(base + 256 + 128 * py, 128, c1_ref[4 * g + 1, :64])
        else:
            rT.wait_recv()
            acc(base + 128 * px + 64 * py, 64, c1_ref[4 * g + 2, :32])
            rB.wait_recv()
            acc(base + 256 + 128 * py + 64 * px, 64,
                c1_ref[4 * g + 3, :32])

    def baseR(g):
        return 512 * g + 128 * px + 64 * py

    def baseL(g):
        return 512 * g + 256 + 128 * py + 64 * px

    def p2_start(g, s):
        off = 16 * g + 4 + 2 * s
        if s == 0:
            rT = xfer(baseR(g) + 32 * (1 - t), 32, c2_ref.at[2 * g],
                      off, pz1)
            rB = xfer(baseL(g) + 32 * (1 - t), 32, c2_ref.at[2 * g + 1],
                      off + 1, pz1)
        else:
            rT = xfer(baseR(g) + 32 * t + 16 * (1 - u), 16,
                      c3_ref.at[2 * g], off, pz2)
            rB = xfer(baseL(g) + 32 * t + 16 * (1 - u), 16,
                      c3_ref.at[2 * g + 1], off + 1, pz2)
        return rT, rB

    def p2_finish(g, s, rT, rB):
        if s == 0:
            rT.wait_recv()
            acc(baseR(g) + 32 * t, 32, c2_ref[2 * g])
            rB.wait_recv()
            acc(baseL(g) + 32 * t, 32, c2_ref[2 * g + 1])
        else:
            rT.wait_recv()
            acc(baseR(g) + 32 * t + 16 * u, 16, c3_ref[2 * g])
            rB.wait_recv()
            acc(baseL(g) + 32 * t + 16 * u, 16, c3_ref[2 * g + 1])

    def p3a_start(g, s):
        off = 16 * g + 8 + 2 * s
        if s == 0:
            qT = baseR(g) + 32 * t + 16 * u
            qB = baseL(g) + 32 * t + 16 * u
            rT = xfer(qT, 16, c4a_ref.at[2 * g], off, pz2)
            rB = xfer(qB, 16, c4a_ref.at[2 * g + 1], off + 1, pz2)
        else:
            hT = baseR(g) + 32 * t
            hB = baseL(g) + 32 * t
            rT = xfer(hT, 32, c4b_ref.at[2 * g], off, pz1)
            rB = xfer(hB, 32, c4b_ref.at[2 * g + 1], off + 1, pz1)
        return rT, rB

    def p3a_finish(g, s, rT, rB):
        if s == 0:
            rT.wait_recv()
            store(baseR(g) + 32 * t + 16 * (1 - u), 16, c4a_ref[2 * g])
            rB.wait_recv()
            store(baseL(g) + 32 * t + 16 * (1 - u), 16, c4a_ref[2 * g + 1])
        else:
            rT.wait_recv()
            store(baseR(g) + 32 * (1 - t), 32, c4b_ref[2 * g])
            rB.wait_recv()
            store(baseL(g) + 32 * (1 - t), 32, c4b_ref[2 * g + 1])

    def p3b_start(g, s):
        base = 512 * g
        off = 16 * g + 12 + 2 * s
        if s == 0:
            rT = xfer(baseR(g), 64, c5_ref.at[4 * g, pl.ds(0, 32)],
                      off, yp)
            rB = xfer(baseL(g), 64, c5_ref.at[4 * g + 1, pl.ds(0, 32)],
                      off + 1, xp)
        else:
            rT = xfer(base + 128 * px, 128,
                      c5_ref.at[4 * g + 2, pl.ds(0, 64)], off, xp)
            rB = xfer(base + 256 + 128 * py, 128,
                      c5_ref.at[4 * g + 3, pl.ds(0, 64)], off + 1, yp)
        return rT, rB

    def p3b_finish(g, s, rT, rB):
        base = 512 * g
        if s == 0:
            rT.wait_recv()
            store(base + 128 * px + 64 * (1 - py), 64,
                  c5_ref[4 * g, :32])
            rB.wait_recv()
            store(base + 256 + 128 * py + 64 * (1 - px), 64,
                  c5_ref[4 * g + 1, :32])
        else:
            rT.wait_recv()
            store(base + 128 * (1 - px), 128, c5_ref[4 * g + 2, :64])
            rB.wait_recv()
            store(base + 256 + 128 * (1 - py), 128, c5_ref[4 * g + 3, :64])

    cp_k.wait()
    cp_v.wait()
    qkv_group(0)
    attn_batch(0)
    attn_batch(1)
    wo_group(0)

    barrier_sem = pltpu.get_barrier_semaphore()
    for nbr in (xp, yp, pz1, pz2):
        pl.semaphore_signal(
            barrier_sem, inc=1,
            device_id=(nbr,), device_id_type=pl.DeviceIdType.MESH)
    pl.semaphore_wait(barrier_sem, 4)

    r = p1_start(0, 0)
    qkv_group(1)
    attn_batch(2)
    p1_finish(0, 0, *r)
    step_done()
    r = p1_start(0, 1)
    attn_batch(3)
    wo_group(1)
    p1_finish(0, 1, *r)
    step_done()

    a = p2_start(0, 0); b = p1_start(1, 0)
    p2_finish(0, 0, *a); p1_finish(1, 0, *b)
    step_done()

    a = p2_start(0, 1); b = p1_start(1, 1)
    p2_finish(0, 1, *a); p1_finish(1, 1, *b)
    step_done()

    a = p3a_start(0, 0); b = p2_start(1, 0)
    p3a_finish(0, 0, *a); p2_finish(1, 0, *b)
    step_done()

    a = p3a_start(0, 1); b = p2_start(1, 1)
    p3a_finish(0, 1, *a); p2_finish(1, 1, *b)
    step_done()

    a = p3b_start(0, 0); b = p3a_start(1, 0)
    p3b_finish(0, 0, *a); p3a_finish(1, 0, *b)
    step_done()

    a = p3b_start(0, 1); b = p3a_start(1, 1)
    p3b_finish(0, 1, *a); p3a_finish(1, 1, *b)
    step_done()

    b = p3b_start(1, 0)
    p3b_finish(1, 0, *b)
    step_done()
    b = p3b_start(1, 1)
    p3b_finish(1, 1, *b)
    step_done(keep=0)


def kernel(x, Wq, Wo, Wk, Wv):
    reduced = pl.pallas_call(
        _fused_body,
        out_shape=jax.ShapeDtypeStruct((T, D), jnp.float32),
        in_specs=[
            pl.BlockSpec(memory_space=pltpu.VMEM),
            pl.BlockSpec(memory_space=pltpu.VMEM),
            pl.BlockSpec(memory_space=pltpu.VMEM),
            pl.BlockSpec(memory_space=pltpu.MemorySpace.HBM),
            pl.BlockSpec(memory_space=pltpu.MemorySpace.HBM),
        ],
        out_specs=pl.BlockSpec(memory_space=pltpu.VMEM),
        scratch_shapes=[
            pltpu.VMEM((D, KV_COLS), jnp.float32),
            pltpu.VMEM((D, KV_COLS), jnp.float32),
            pltpu.VMEM((T, D), jnp.float32),
            pltpu.VMEM((T, KV_COLS), jnp.float32),
            pltpu.VMEM((T, KV_COLS), jnp.float32),
            pltpu.VMEM((T, D), jnp.float32),
            pltpu.VMEM((8, 64, D), WIRE),
            pltpu.VMEM((4, 16, D), WIRE),
            pltpu.VMEM((4, 8, D), WIRE),
            pltpu.VMEM((4, 8, D), WIRE),
            pltpu.VMEM((4, 16, D), WIRE),
            pltpu.VMEM((8, 64, D), WIRE),
            pltpu.VMEM((960, D), WIRE),
            pltpu.SemaphoreType.DMA((2,)),
            pltpu.SemaphoreType.DMA((32,)),
            pltpu.SemaphoreType.DMA((32,)),
        ],
        compiler_params=pltpu.CompilerParams(collective_id=0),
    )(x, Wq, Wo, Wk, Wv)

    return reduced.reshape(B, SQ, D)
